# Initial kernel scaffold; baseline (speedup 1.0000x reference)
import jax
import jax.numpy as jnp
from jax.experimental import pallas as pl


def kernel(agent_obs, hideout_obs, timestep_obs, num_agents, edge_index, tc1_w1, tc1_b1, tc1_w2, tc1_b2, tc1_w3, tc1_b3, cheb_w, cheb_b, tc2_w1, tc2_b1, tc2_w2, tc2_b2, tc2_w3, tc2_b3, bn_gamma, bn_beta, lin_w, lin_b):
    raise NotImplementedError("write your pallas kernel here")


# speedup vs baseline: 18.0903x; 18.0903x over previous
import jax
import jax.numpy as jnp
from jax.experimental import pallas as pl

_N = 83
_E = 3403
_B = 32
_T = 50
_CIN = 32
_CH = 16
_COUT = 32
_P = 12
_EPS = 1e-5
_T1 = _T - 2
_T2 = _T - 4
_BN_CNT = _B * _T2 * _COUT


def _lhat_kernel(ei_ref, eit_ref, lhatT_ref):
    src_row = ei_ref[0:1, :]
    dst_row = ei_ref[1:2, :]
    src_col = eit_ref[:, 0:1]
    dst_col = eit_ref[:, 1:2]
    w_row = (src_row != dst_row).astype(jnp.float32)
    iota_ne = jax.lax.broadcasted_iota(jnp.int32, (_N, _E), 0)
    iota_en = jax.lax.broadcasted_iota(jnp.int32, (_E, _N), 1)
    oh_src_ne = (iota_ne == src_row).astype(jnp.float32)
    oh_dst_en = (iota_en == dst_col).astype(jnp.float32)
    deg = jnp.sum(oh_src_ne * w_row, axis=1, keepdims=True)
    dinv = jnp.where(deg > 0, jax.lax.rsqrt(jnp.where(deg > 0, deg, 1.0)), 0.0)
    dinv_src = jnp.sum(oh_src_ne * dinv, axis=0, keepdims=True)
    oh_dst_ne = (iota_ne == dst_row).astype(jnp.float32)
    dinv_dst = jnp.sum(oh_dst_ne * dinv, axis=0, keepdims=True)
    norm = -dinv_src * w_row * dinv_dst
    lhatT_ref[...] = jnp.dot(oh_src_ne * norm, oh_dst_en,
                             preferred_element_type=jnp.float32)


def _main_kernel(x_ref, lhatT_ref, w1_ref, b1_ref, w2_ref, b2_ref, w3_ref,
                 b3_ref, cw0_ref, cw1_ref, cw2_ref, cb_ref, v1_ref, c1_ref,
                 v2_ref, c2_ref, v3_ref, c3_ref, sum_ref, sumsq_ref, last_ref):
    x = x_ref[0]
    x0 = x[0:_T1].reshape(_T1 * _N, _CIN)
    x1 = x[1:_T1 + 1].reshape(_T1 * _N, _CIN)
    x2 = x[2:_T1 + 2].reshape(_T1 * _N, _CIN)
    xc = jnp.concatenate([x0, x1, x2], axis=1)
    p = xc @ w1_ref[...] + b1_ref[...]
    q = jax.nn.sigmoid(xc @ w2_ref[...] + b2_ref[...])
    r = xc @ w3_ref[...] + b3_ref[...]
    f0 = jax.nn.relu(p * q + r)

    lhatT = lhatT_ref[...]
    zc0 = f0.reshape(_T1, _N, _CH).transpose(0, 2, 1).reshape(_T1 * _CH, _N)
    a1 = jnp.dot(zc0, lhatT, preferred_element_type=jnp.float32)
    a2 = 2.0 * jnp.dot(a1, lhatT, preferred_element_type=jnp.float32) - zc0
    f1 = a1.reshape(_T1, _CH, _N).transpose(0, 2, 1).reshape(_T1 * _N, _CH)
    f2 = a2.reshape(_T1, _CH, _N).transpose(0, 2, 1).reshape(_T1 * _N, _CH)
    out = f0 @ cw0_ref[...] + f1 @ cw1_ref[...] + f2 @ cw2_ref[...] + cb_ref[...]
    tg = jax.nn.relu(out).reshape(_T1, _N, _CH)

    y0 = tg[0:_T2].reshape(_T2 * _N, _CH)
    y1 = tg[1:_T2 + 1].reshape(_T2 * _N, _CH)
    y2 = tg[2:_T2 + 2].reshape(_T2 * _N, _CH)
    yc = jnp.concatenate([y0, y1, y2], axis=1)
    p2 = yc @ v1_ref[...] + c1_ref[...]
    q2 = jax.nn.sigmoid(yc @ v2_ref[...] + c2_ref[...])
    r2 = yc @ v3_ref[...] + c3_ref[...]
    h2 = jax.nn.relu(p2 * q2 + r2)
    h2_3 = h2.reshape(_T2, _N, _COUT)

    s_tn = jnp.sum(h2_3, axis=2)
    ss_tn = jnp.sum(h2_3 * h2_3, axis=2)
    sum_ref[0] = jnp.sum(s_tn, axis=0, keepdims=True)
    sumsq_ref[0] = jnp.sum(ss_tn, axis=0, keepdims=True)
    last_ref[0] = h2_3[_T2 - 1]


def _final_kernel(sums_ref, sumsq_ref, last_ref, gamma_ref, beta_ref,
                  linw_ref, linb_ref, hide_ref, ts_ref, out_ref):
    s = jnp.sum(sums_ref[...], axis=1, keepdims=True)
    ss = jnp.sum(sumsq_ref[...], axis=1, keepdims=True)
    mean = s / _BN_CNT
    var = ss / _BN_CNT - mean * mean
    inv = gamma_ref[...] * jax.lax.rsqrt(var + _EPS)
    shift = beta_ref[...] - mean * inv
    last = last_ref[...]
    h = jax.nn.relu(last * inv[None] + shift[None])
    hl = h.reshape(_B * _N, _COUT) @ linw_ref[...] + linb_ref[...]
    pooled = jnp.mean(hl.reshape(_B, _N, _P), axis=1)
    out_ref[:, 0:_P] = pooled
    out_ref[:, _P:_P + 2] = hide_ref[...]
    out_ref[:, _P + 2:_P + 3] = ts_ref[...]


def kernel(agent_obs, hideout_obs, timestep_obs, num_agents, edge_index,
           tc1_w1, tc1_b1, tc1_w2, tc1_b2, tc1_w3, tc1_b3, cheb_w, cheb_b,
           tc2_w1, tc2_b1, tc2_w2, tc2_b2, tc2_w3, tc2_b3, bn_gamma, bn_beta,
           lin_w, lin_b):
    f32 = jnp.float32

    lhatT = pl.pallas_call(
        _lhat_kernel,
        out_shape=jax.ShapeDtypeStruct((_N, _N), f32),
    )(edge_index, edge_index.T)

    def cat_w(w):
        return jnp.concatenate([w[:, :, 0, k].T for k in range(3)], axis=0)

    w1, w2, w3 = cat_w(tc1_w1), cat_w(tc1_w2), cat_w(tc1_w3)
    b1, b2, b3 = (tc1_b1.reshape(1, _CH), tc1_b2.reshape(1, _CH),
                  tc1_b3.reshape(1, _CH))
    v1, v2, v3 = cat_w(tc2_w1), cat_w(tc2_w2), cat_w(tc2_w3)
    c1, c2, c3 = (tc2_b1.reshape(1, _COUT), tc2_b2.reshape(1, _COUT),
                  tc2_b3.reshape(1, _COUT))
    cb = cheb_b.reshape(1, _CH)

    full = lambda shape: pl.BlockSpec(shape, lambda b: (0,) * len(shape))
    sums, sumsq, last = pl.pallas_call(
        _main_kernel,
        grid=(_B,),
        in_specs=[
            pl.BlockSpec((1, _T, _N, _CIN), lambda b: (b, 0, 0, 0)),
            full((_N, _N)),
            full((3 * _CIN, _CH)), full((1, _CH)),
            full((3 * _CIN, _CH)), full((1, _CH)),
            full((3 * _CIN, _CH)), full((1, _CH)),
            full((_CH, _CH)), full((_CH, _CH)), full((_CH, _CH)),
            full((1, _CH)),
            full((3 * _CH, _COUT)), full((1, _COUT)),
            full((3 * _CH, _COUT)), full((1, _COUT)),
            full((3 * _CH, _COUT)), full((1, _COUT)),
        ],
        out_specs=[
            pl.BlockSpec((1, 1, _N), lambda b: (b, 0, 0)),
            pl.BlockSpec((1, 1, _N), lambda b: (b, 0, 0)),
            pl.BlockSpec((1, _N, _COUT), lambda b: (b, 0, 0)),
        ],
        out_shape=[
            jax.ShapeDtypeStruct((_B, 1, _N), f32),
            jax.ShapeDtypeStruct((_B, 1, _N), f32),
            jax.ShapeDtypeStruct((_B, _N, _COUT), f32),
        ],
    )(agent_obs, lhatT, w1, b1, w2, b2, w3, b3,
      cheb_w[0], cheb_w[1], cheb_w[2], cb,
      v1, c1, v2, c2, v3, c3)

    out = pl.pallas_call(
        _final_kernel,
        out_shape=jax.ShapeDtypeStruct((_B, _P + 3), f32),
    )(sums.reshape(_B, _N).T, sumsq.reshape(_B, _N).T, last,
      bn_gamma.reshape(_N, 1), bn_beta.reshape(_N, 1),
      lin_w.T, lin_b.reshape(1, _P), hideout_obs, timestep_obs)

    return out


# speedup vs baseline: 30.7605x; 1.7004x over previous
import jax
import jax.numpy as jnp
from jax.experimental import pallas as pl

_N = 83
_NP = 96
_E = 3403
_B = 32
_T = 50
_CIN = 32
_CH = 16
_COUT = 32
_P = 12
_EPS = 1e-5
_T1 = _T - 2
_T2 = _T - 4
_BN_CNT = _B * _T2 * _COUT


def _lhat_kernel(ei_ref, eit_ref, lhatT_ref):
    src_row = ei_ref[0:1, :]
    dst_row = ei_ref[1:2, :]
    dst_col = eit_ref[:, 1:2]
    w_row = (src_row != dst_row).astype(jnp.float32)
    iota_ne = jax.lax.broadcasted_iota(jnp.int32, (_NP, _E), 0)
    iota_en = jax.lax.broadcasted_iota(jnp.int32, (_E, _NP), 1)
    oh_src_ne = (iota_ne == src_row).astype(jnp.float32)
    oh_dst_en = (iota_en == dst_col).astype(jnp.float32)
    deg = jnp.sum(oh_src_ne * w_row, axis=1, keepdims=True)
    dinv = jnp.where(deg > 0, jax.lax.rsqrt(jnp.where(deg > 0, deg, 1.0)), 0.0)
    dinv_src = jnp.sum(oh_src_ne * dinv, axis=0, keepdims=True)
    oh_dst_ne = (iota_ne == dst_row).astype(jnp.float32)
    dinv_dst = jnp.sum(oh_dst_ne * dinv, axis=0, keepdims=True)
    norm = -dinv_src * w_row * dinv_dst
    lhatT_ref[...] = jnp.dot(oh_src_ne * norm, oh_dst_en,
                             preferred_element_type=jnp.float32)


def _main_kernel(x_ref, lhatT_ref, w123_ref, b123_ref, cw0_ref, cw1_ref,
                 cw2_ref, cb_ref, v123_ref, c123_ref,
                 sum_ref, sumsq_ref, last_ref):
    x = x_ref[0]
    x0 = x[0:_T1].reshape(_T1 * _NP, _CIN)
    x1 = x[1:_T1 + 1].reshape(_T1 * _NP, _CIN)
    x2 = x[2:_T1 + 2].reshape(_T1 * _NP, _CIN)
    xc = jnp.concatenate([x0, x1, x2], axis=1)
    g = xc @ w123_ref[...] + b123_ref[...]
    p = g[:, 0:_CH]
    q = g[:, _CH:2 * _CH]
    r = g[:, 2 * _CH:3 * _CH]
    f0 = jax.nn.relu(p * jax.nn.sigmoid(q) + r)

    lhatT = lhatT_ref[...]
    zc0 = f0.reshape(_T1, _NP, _CH).transpose(0, 2, 1).reshape(_T1 * _CH, _NP)
    a1 = jnp.dot(zc0, lhatT, preferred_element_type=jnp.float32)
    a2 = 2.0 * jnp.dot(a1, lhatT, preferred_element_type=jnp.float32) - zc0
    f1 = a1.reshape(_T1, _CH, _NP).transpose(0, 2, 1).reshape(_T1 * _NP, _CH)
    f2 = a2.reshape(_T1, _CH, _NP).transpose(0, 2, 1).reshape(_T1 * _NP, _CH)
    out = f0 @ cw0_ref[...] + f1 @ cw1_ref[...] + f2 @ cw2_ref[...] + cb_ref[...]
    tg = jax.nn.relu(out).reshape(_T1, _NP, _CH)

    y0 = tg[0:_T2].reshape(_T2 * _NP, _CH)
    y1 = tg[1:_T2 + 1].reshape(_T2 * _NP, _CH)
    y2 = tg[2:_T2 + 2].reshape(_T2 * _NP, _CH)
    yc = jnp.concatenate([y0, y1, y2], axis=1)
    g2 = yc @ v123_ref[...] + c123_ref[...]
    p2 = g2[:, 0:_COUT]
    q2 = g2[:, _COUT:2 * _COUT]
    r2 = g2[:, 2 * _COUT:3 * _COUT]
    h2 = jax.nn.relu(p2 * jax.nn.sigmoid(q2) + r2)
    h2_3 = h2.reshape(_T2, _NP, _COUT)

    s_tn = jnp.sum(h2_3, axis=2)
    ss_tn = jnp.sum(h2_3 * h2_3, axis=2)
    sum_ref[0] = jnp.sum(s_tn, axis=0, keepdims=True)
    sumsq_ref[0] = jnp.sum(ss_tn, axis=0, keepdims=True)
    last_ref[0] = h2_3[_T2 - 1]


def _final_kernel(sums_ref, sumsq_ref, last_ref, gamma_ref, beta_ref,
                  linw_ref, linb_ref, hide_ref, ts_ref, out_ref):
    s = jnp.sum(sums_ref[...], axis=1, keepdims=True)
    ss = jnp.sum(sumsq_ref[...], axis=1, keepdims=True)
    mean = s / _BN_CNT
    var = ss / _BN_CNT - mean * mean
    inv = gamma_ref[...] * jax.lax.rsqrt(jnp.abs(var) + _EPS)
    shift = beta_ref[...] - mean * inv
    last = last_ref[...]
    h = jax.nn.relu(last * inv[None] + shift[None])
    hl = h.reshape(_B * _NP, _COUT) @ linw_ref[...]
    pooled = jnp.sum(hl.reshape(_B, _NP, _P), axis=1) * (1.0 / _N) + linb_ref[...]
    out_ref[:, 0:_P] = pooled
    out_ref[:, _P:_P + 2] = hide_ref[...]
    out_ref[:, _P + 2:_P + 3] = ts_ref[...]


def kernel(agent_obs, hideout_obs, timestep_obs, num_agents, edge_index,
           tc1_w1, tc1_b1, tc1_w2, tc1_b2, tc1_w3, tc1_b3, cheb_w, cheb_b,
           tc2_w1, tc2_b1, tc2_w2, tc2_b2, tc2_w3, tc2_b3, bn_gamma, bn_beta,
           lin_w, lin_b):
    f32 = jnp.float32

    lhatT = pl.pallas_call(
        _lhat_kernel,
        out_shape=jax.ShapeDtypeStruct((_NP, _NP), f32),
    )(edge_index, edge_index.T)

    def cat_w(w):
        return jnp.concatenate([w[:, :, 0, k].T for k in range(3)], axis=0)

    w123 = jnp.concatenate([cat_w(tc1_w1), cat_w(tc1_w2), cat_w(tc1_w3)], axis=1)
    b123 = jnp.concatenate([tc1_b1, tc1_b2, tc1_b3]).reshape(1, 3 * _CH)
    v123 = jnp.concatenate([cat_w(tc2_w1), cat_w(tc2_w2), cat_w(tc2_w3)], axis=1)
    c123 = jnp.concatenate([tc2_b1, tc2_b2, tc2_b3]).reshape(1, 3 * _COUT)
    cb = cheb_b.reshape(1, _CH)

    x_pad = jnp.pad(agent_obs, ((0, 0), (0, 0), (0, _NP - _N), (0, 0)))

    full = lambda shape: pl.BlockSpec(shape, lambda b: (0,) * len(shape))
    sums, sumsq, last = pl.pallas_call(
        _main_kernel,
        grid=(_B,),
        in_specs=[
            pl.BlockSpec((1, _T, _NP, _CIN), lambda b: (b, 0, 0, 0)),
            full((_NP, _NP)),
            full((3 * _CIN, 3 * _CH)), full((1, 3 * _CH)),
            full((_CH, _CH)), full((_CH, _CH)), full((_CH, _CH)),
            full((1, _CH)),
            full((3 * _CH, 3 * _COUT)), full((1, 3 * _COUT)),
        ],
        out_specs=[
            pl.BlockSpec((1, 1, _NP), lambda b: (b, 0, 0)),
            pl.BlockSpec((1, 1, _NP), lambda b: (b, 0, 0)),
            pl.BlockSpec((1, _NP, _COUT), lambda b: (b, 0, 0)),
        ],
        out_shape=[
            jax.ShapeDtypeStruct((_B, 1, _NP), f32),
            jax.ShapeDtypeStruct((_B, 1, _NP), f32),
            jax.ShapeDtypeStruct((_B, _NP, _COUT), f32),
        ],
    )(x_pad, lhatT, w123, b123,
      cheb_w[0], cheb_w[1], cheb_w[2], cb, v123, c123)

    gamma_pad = jnp.pad(bn_gamma, (0, _NP - _N)).reshape(_NP, 1)
    beta_pad = jnp.pad(bn_beta, (0, _NP - _N)).reshape(_NP, 1)

    out = pl.pallas_call(
        _final_kernel,
        out_shape=jax.ShapeDtypeStruct((_B, _P + 3), f32),
    )(sums.reshape(_B, _NP).T, sumsq.reshape(_B, _NP).T, last,
      gamma_pad, beta_pad,
      lin_w.T, lin_b.reshape(1, _P), hideout_obs, timestep_obs)

    return out


# speedup vs baseline: 34.0988x; 1.1085x over previous
import jax
import jax.numpy as jnp
from jax.experimental import pallas as pl

_N = 83
_NP = 96
_E = 3403
_B = 32
_T = 50
_CIN = 32
_CH = 16
_COUT = 32
_P = 12
_EPS = 1e-5
_T1 = _T - 2
_T2 = _T - 4
_BN_CNT = _B * _T2 * _COUT


def _lhat_kernel(ei_ref, eit_ref, lhatT_ref):
    src_row = ei_ref[0:1, :]
    dst_row = ei_ref[1:2, :]
    dst_col = eit_ref[:, 1:2]
    w_row = (src_row != dst_row).astype(jnp.float32)
    iota_ne = jax.lax.broadcasted_iota(jnp.int32, (_NP, _E), 0)
    iota_en = jax.lax.broadcasted_iota(jnp.int32, (_E, _NP), 1)
    oh_src_ne = (iota_ne == src_row).astype(jnp.float32)
    oh_dst_en = (iota_en == dst_col).astype(jnp.float32)
    deg = jnp.sum(oh_src_ne * w_row, axis=1, keepdims=True)
    dinv = jnp.where(deg > 0, jax.lax.rsqrt(jnp.where(deg > 0, deg, 1.0)), 0.0)
    dinv_src = jnp.sum(oh_src_ne * dinv, axis=0, keepdims=True)
    oh_dst_ne = (iota_ne == dst_row).astype(jnp.float32)
    dinv_dst = jnp.sum(oh_dst_ne * dinv, axis=0, keepdims=True)
    norm = -dinv_src * w_row * dinv_dst
    lhatT_ref[...] = jnp.dot(oh_src_ne * norm, oh_dst_en,
                             preferred_element_type=jnp.float32)


def _main_kernel(x_ref, lhatT_ref, wp_ref, wq_ref, wr_ref, b1_ref,
                 cw0_ref, cw1_ref, cw2_ref, cb_ref,
                 vp_ref, vq_ref, vr_ref, b2_ref,
                 sum_ref, sumsq_ref, last_ref):
    x = x_ref[0]
    xf = x.reshape(_T * _NP, _CIN)
    x0 = xf[0:_T1 * _NP]
    x1 = xf[_NP:(_T1 + 1) * _NP]
    x2 = xf[2 * _NP:(_T1 + 2) * _NP]
    xc = jnp.concatenate([x0, x1, x2], axis=1)
    p = xc @ wp_ref[...] + b1_ref[0:1]
    q = xc @ wq_ref[...] + b1_ref[1:2]
    r = xc @ wr_ref[...] + b1_ref[2:3]
    f0 = jax.nn.relu(p * jax.nn.sigmoid(q) + r)

    lhatT = lhatT_ref[...]
    zc0 = f0.reshape(_T1, _NP, _CH).transpose(0, 2, 1).reshape(_T1 * _CH, _NP)
    a1 = jnp.dot(zc0, lhatT, preferred_element_type=jnp.float32)
    a2 = 2.0 * jnp.dot(a1, lhatT, preferred_element_type=jnp.float32) - zc0
    f1 = a1.reshape(_T1, _CH, _NP).transpose(0, 2, 1).reshape(_T1 * _NP, _CH)
    f2 = a2.reshape(_T1, _CH, _NP).transpose(0, 2, 1).reshape(_T1 * _NP, _CH)
    out = f0 @ cw0_ref[...] + f1 @ cw1_ref[...] + f2 @ cw2_ref[...] + cb_ref[...]
    tg = jax.nn.relu(out)

    y0 = tg[0:_T2 * _NP]
    y1 = tg[_NP:(_T2 + 1) * _NP]
    y2 = tg[2 * _NP:(_T2 + 2) * _NP]
    yc = jnp.concatenate([y0, y1, y2], axis=1)
    p2 = yc @ vp_ref[...] + b2_ref[0:1]
    q2 = yc @ vq_ref[...] + b2_ref[1:2]
    r2 = yc @ vr_ref[...] + b2_ref[2:3]
    h2 = jax.nn.relu(p2 * jax.nn.sigmoid(q2) + r2)
    h2_3 = h2.reshape(_T2, _NP, _COUT)

    s_tn = jnp.sum(h2_3, axis=2)
    ss_tn = jnp.sum(h2_3 * h2_3, axis=2)
    sum_ref[0] = jnp.sum(s_tn, axis=0, keepdims=True)
    sumsq_ref[0] = jnp.sum(ss_tn, axis=0, keepdims=True)
    last_ref[0] = h2_3[_T2 - 1]


def _final_kernel(sums_ref, sumsq_ref, last_ref, gamma_ref, beta_ref,
                  linw_ref, linb_ref, hide_ref, ts_ref, out_ref):
    s = jnp.sum(sums_ref[...], axis=1, keepdims=True)
    ss = jnp.sum(sumsq_ref[...], axis=1, keepdims=True)
    mean = s / _BN_CNT
    var = ss / _BN_CNT - mean * mean
    inv = gamma_ref[...] * jax.lax.rsqrt(jnp.abs(var) + _EPS)
    shift = beta_ref[...] - mean * inv
    last = last_ref[...]
    h = jax.nn.relu(last * inv[None] + shift[None])
    hl = h.reshape(_B * _NP, _COUT) @ linw_ref[...]
    pooled = jnp.sum(hl.reshape(_B, _NP, _P), axis=1) * (1.0 / _N) + linb_ref[...]
    out_ref[:, 0:_P] = pooled
    out_ref[:, _P:_P + 2] = hide_ref[...]
    out_ref[:, _P + 2:_P + 3] = ts_ref[...]


def kernel(agent_obs, hideout_obs, timestep_obs, num_agents, edge_index,
           tc1_w1, tc1_b1, tc1_w2, tc1_b2, tc1_w3, tc1_b3, cheb_w, cheb_b,
           tc2_w1, tc2_b1, tc2_w2, tc2_b2, tc2_w3, tc2_b3, bn_gamma, bn_beta,
           lin_w, lin_b):
    f32 = jnp.float32

    lhatT = pl.pallas_call(
        _lhat_kernel,
        out_shape=jax.ShapeDtypeStruct((_NP, _NP), f32),
    )(edge_index, edge_index.T)

    def cat_w(w):
        return jnp.concatenate([w[:, :, 0, k].T for k in range(3)], axis=0)

    wp, wq, wr = cat_w(tc1_w1), cat_w(tc1_w2), cat_w(tc1_w3)
    b1 = jnp.stack([tc1_b1, tc1_b2, tc1_b3])
    vp, vq, vr = cat_w(tc2_w1), cat_w(tc2_w2), cat_w(tc2_w3)
    b2 = jnp.stack([tc2_b1, tc2_b2, tc2_b3])
    cb = cheb_b.reshape(1, _CH)

    x_pad = jnp.pad(agent_obs, ((0, 0), (0, 0), (0, _NP - _N), (0, 0)))

    full = lambda shape: pl.BlockSpec(shape, lambda b: (0,) * len(shape))
    sums, sumsq, last = pl.pallas_call(
        _main_kernel,
        grid=(_B,),
        in_specs=[
            pl.BlockSpec((1, _T, _NP, _CIN), lambda b: (b, 0, 0, 0)),
            full((_NP, _NP)),
            full((3 * _CIN, _CH)), full((3 * _CIN, _CH)),
            full((3 * _CIN, _CH)), full((3, _CH)),
            full((_CH, _CH)), full((_CH, _CH)), full((_CH, _CH)),
            full((1, _CH)),
            full((3 * _CH, _COUT)), full((3 * _CH, _COUT)),
            full((3 * _CH, _COUT)), full((3, _COUT)),
        ],
        out_specs=[
            pl.BlockSpec((1, 1, _NP), lambda b: (b, 0, 0)),
            pl.BlockSpec((1, 1, _NP), lambda b: (b, 0, 0)),
            pl.BlockSpec((1, _NP, _COUT), lambda b: (b, 0, 0)),
        ],
        out_shape=[
            jax.ShapeDtypeStruct((_B, 1, _NP), f32),
            jax.ShapeDtypeStruct((_B, 1, _NP), f32),
            jax.ShapeDtypeStruct((_B, _NP, _COUT), f32),
        ],
    )(x_pad, lhatT, wp, wq, wr, b1,
      cheb_w[0], cheb_w[1], cheb_w[2], cb, vp, vq, vr, b2)

    gamma_pad = jnp.pad(bn_gamma, (0, _NP - _N)).reshape(_NP, 1)
    beta_pad = jnp.pad(bn_beta, (0, _NP - _N)).reshape(_NP, 1)

    out = pl.pallas_call(
        _final_kernel,
        out_shape=jax.ShapeDtypeStruct((_B, _P + 3), f32),
    )(sums.reshape(_B, _NP).T, sumsq.reshape(_B, _NP).T, last,
      gamma_pad, beta_pad,
      lin_w.T, lin_b.reshape(1, _P), hideout_obs, timestep_obs)

    return out
